# baseline (speedup 1.0000x reference)
import jax
import jax.numpy as jnp
from jax.experimental import pallas as pl
from jax.experimental.pallas import tpu as pltpu

E, D, F, T = 8, 768, 2048, 2048
FS = 4
FC = F // FS


def _moe_kernel(xb_ref, W1_ref, b1_ref, W2_ref, b2_ref, Wr_ref, br_ref,
                out_ref, w_ref):
    e = pl.program_id(0)

    @pl.when(e == 0)
    def _():
        logits = jnp.dot(xb_ref[...], Wr_ref[...].astype(jnp.bfloat16),
                         preferred_element_type=jnp.float32) + br_ref[...]
        m = jnp.max(logits, axis=-1, keepdims=True)
        p = jnp.exp(logits - m)
        w_ref[...] = p / jnp.sum(p, axis=-1, keepdims=True)

    xb = xb_ref[...]
    b1 = b1_ref[0]
    b2 = b2_ref[0]
    lane = jax.lax.broadcasted_iota(jnp.int32, (T, E), 1)

    acc = None
    for j in range(FS):
        fs = slice(j * FC, (j + 1) * FC)
        h = jnp.dot(xb, W1_ref[0][:, fs].astype(jnp.bfloat16),
                    preferred_element_type=jnp.float32) + b1[:, fs]
        g = (h * (1.0 + jax.lax.erf(h * 0.7071067811865476))
             ).astype(jnp.bfloat16)
        part = jnp.dot(g, W2_ref[0][fs, :].astype(jnp.bfloat16),
                       preferred_element_type=jnp.float32)
        acc = part if acc is None else acc + part

    wc = jnp.sum(jnp.where(lane == e, w_ref[...], 0.0), axis=1,
                 keepdims=True)
    contrib = (0.5 * wc) * acc + wc * b2

    @pl.when(e == 0)
    def _():
        out_ref[...] = contrib

    @pl.when(e > 0)
    def _():
        out_ref[...] = out_ref[...] + contrib


def kernel(x, W1, b1, W2, b2, Wr, br):
    xb = x.astype(jnp.bfloat16)
    br2 = br.reshape(1, E)
    b1r = b1.reshape(E, 1, F)
    b2r = b2.reshape(E, 1, D)
    return pl.pallas_call(
        _moe_kernel,
        grid=(E,),
        in_specs=[
            pl.BlockSpec((T, D), lambda e: (0, 0)),
            pl.BlockSpec((1, D, F), lambda e: (e, 0, 0)),
            pl.BlockSpec((1, 1, F), lambda e: (e, 0, 0)),
            pl.BlockSpec((1, F, D), lambda e: (e, 0, 0)),
            pl.BlockSpec((1, 1, D), lambda e: (e, 0, 0)),
            pl.BlockSpec((D, E), lambda e: (0, 0)),
            pl.BlockSpec((1, E), lambda e: (0, 0)),
        ],
        out_specs=pl.BlockSpec((T, D), lambda e: (0, 0)),
        out_shape=jax.ShapeDtypeStruct((T, D), jnp.float32),
        scratch_shapes=[pltpu.VMEM((T, E), jnp.float32)],
        compiler_params=pltpu.CompilerParams(
            dimension_semantics=("arbitrary",),
        ),
    )(xb, W1, b1r, W2, b2r, Wr, br2)
